# baseline (speedup 1.0000x reference)
import functools

import jax
import jax.numpy as jnp
from jax import lax
from jax.experimental import pallas as pl
from jax.experimental.pallas import tpu as pltpu
from jax.experimental.pallas import tpu_sc as plsc

VOCAB = 1000000
DIM = 64

NC = 2
NS = 16
NW = NC * NS

GATHER = 128
CHUNK_GATHERS = 4
CHUNK = GATHER * CHUNK_GATHERS


def _body(idx_hbm, table_hbm, out_hbm, idx_v, rows_v, gsem):
    wid = lax.axis_index("s") * NC + lax.axis_index("c")
    n_chunks = idx_hbm.shape[1] // CHUNK_GATHERS
    rows_per_w = idx_hbm.shape[1] * GATHER
    base = wid * rows_per_w

    pltpu.sync_copy(idx_hbm.at[wid], idx_v)

    def chunk(c, _):
        copies = []
        for g in range(CHUNK_GATHERS):
            j = c * CHUNK_GATHERS + g
            copies.append(pltpu.async_copy(
                table_hbm.at[idx_v.at[j]],
                rows_v.at[pl.ds(g * GATHER, GATHER)],
                gsem,
            ))
        for cp in copies:
            cp.wait()
        pltpu.sync_copy(rows_v, out_hbm.at[pl.ds(base + c * CHUNK, CHUNK)])
        return ()

    lax.fori_loop(0, n_chunks, chunk, (), unroll=False)


@jax.jit
def _embed(idx, table):
    n_per_w = idx.shape[0] // NW
    k = n_per_w // GATHER
    idx3 = idx.reshape(NW, k, GATHER)
    mesh = plsc.VectorSubcoreMesh(core_axis_name="c", subcore_axis_name="s")
    out = pl.kernel(
        _body,
        out_type=jax.ShapeDtypeStruct((idx.shape[0], DIM), jnp.float32),
        mesh=mesh,
        scratch_types=[
            pltpu.VMEM((k, GATHER), jnp.int32),
            pltpu.VMEM((CHUNK, DIM), jnp.float32),
            pltpu.SemaphoreType.DMA,
        ],
        compiler_params=pltpu.CompilerParams(use_tc_tiling_on_sc=False),
    )(idx3, table)
    return out


def kernel(input_ids, table):
    B, S = input_ids.shape
    flat = input_ids.reshape(B * S).astype(jnp.int32)
    out = _embed(flat, table)
    return out.reshape(B, S, DIM)


# speedup vs baseline: 1.0267x; 1.0267x over previous
import functools

import jax
import jax.numpy as jnp
from jax import lax
from jax.experimental import pallas as pl
from jax.experimental.pallas import tpu as pltpu
from jax.experimental.pallas import tpu_sc as plsc

VOCAB = 1000000
DIM = 64

NC = 2
NS = 16
NW = NC * NS

GATHER = 128
NBUF = 8


def _body(idx_hbm, table_hbm, out_hbm, idx_v, rows_v, *sems):
    gsem = sems[:NBUF]
    wsem = sems[NBUF:]
    wid = lax.axis_index("s") * NC + lax.axis_index("c")
    n = idx_hbm.shape[1]
    base = wid * n * GATHER

    def fire_g(b, c):
        pltpu.async_copy(table_hbm.at[idx_v.at[c]], rows_v.at[b], gsem[b])

    def wait_g(b):
        pltpu.make_async_copy(
            table_hbm.at[pl.ds(0, GATHER)], rows_v.at[b], gsem[b]).wait()

    def fire_w(b, c):
        pltpu.async_copy(
            rows_v.at[b], out_hbm.at[pl.ds(base + c * GATHER, GATHER)],
            wsem[b])

    def wait_w(b):
        pltpu.make_async_copy(
            rows_v.at[b], out_hbm.at[pl.ds(0, GATHER)], wsem[b]).wait()

    pltpu.sync_copy(idx_hbm.at[wid], idx_v)

    for b in range(NBUF):
        fire_g(b, b)

    def outer(i, _):
        for b in range(NBUF):
            c = i * NBUF + b
            wait_g(b)
            fire_w(b, c)
            wait_w(b)
            fire_g(b, c + NBUF)
        return ()

    lax.fori_loop(0, n // NBUF - 1, outer, ())

    for b in range(NBUF):
        c = n - NBUF + b
        wait_g(b)
        fire_w(b, c)
        wait_w(b)


@jax.jit
def _embed(idx, table):
    k = idx.shape[0] // (NW * GATHER)
    idx3 = idx.reshape(NW, k, GATHER)
    mesh = plsc.VectorSubcoreMesh(core_axis_name="c", subcore_axis_name="s")
    out = pl.kernel(
        _body,
        out_type=jax.ShapeDtypeStruct((idx.shape[0], DIM), jnp.float32),
        mesh=mesh,
        scratch_types=(
            [pltpu.VMEM((k, GATHER), jnp.int32),
             pltpu.VMEM((NBUF, GATHER, DIM), jnp.float32)]
            + [pltpu.SemaphoreType.DMA] * (2 * NBUF)
        ),
        compiler_params=pltpu.CompilerParams(use_tc_tiling_on_sc=False),
    )(idx3, table)
    return out


def kernel(input_ids, table):
    B, S = input_ids.shape
    flat = input_ids.reshape(B * S).astype(jnp.int32)
    out = _embed(flat, table)
    return out.reshape(B, S, DIM)


# speedup vs baseline: 1.0267x; 1.0001x over previous
import functools

import jax
import jax.numpy as jnp
from jax import lax
from jax.experimental import pallas as pl
from jax.experimental.pallas import tpu as pltpu
from jax.experimental.pallas import tpu_sc as plsc

VOCAB = 1000000
DIM = 64

NC = 2
NS = 16
NW = NC * NS

GATHER = 512
NBUF = 2


def _body(idx_hbm, table_hbm, out_hbm, idx_v, rows_v, *sems):
    gsem = sems[:NBUF]
    wsem = sems[NBUF:]
    wid = lax.axis_index("s") * NC + lax.axis_index("c")
    n = idx_hbm.shape[1]
    base = wid * n * GATHER

    def fire_g(b, c):
        pltpu.async_copy(table_hbm.at[idx_v.at[c]], rows_v.at[b], gsem[b])

    def wait_g(b):
        pltpu.make_async_copy(
            table_hbm.at[pl.ds(0, GATHER)], rows_v.at[b], gsem[b]).wait()

    def fire_w(b, c):
        pltpu.async_copy(
            rows_v.at[b], out_hbm.at[pl.ds(base + c * GATHER, GATHER)],
            wsem[b])

    def wait_w(b):
        pltpu.make_async_copy(
            rows_v.at[b], out_hbm.at[pl.ds(0, GATHER)], wsem[b]).wait()

    pltpu.sync_copy(idx_hbm.at[wid], idx_v)

    for b in range(NBUF):
        fire_g(b, b)

    def outer(i, _):
        for b in range(NBUF):
            c = i * NBUF + b
            wait_g(b)
            fire_w(b, c)
            wait_w(b)
            fire_g(b, c + NBUF)
        return ()

    lax.fori_loop(0, n // NBUF - 1, outer, ())

    for b in range(NBUF):
        c = n - NBUF + b
        wait_g(b)
        fire_w(b, c)
        wait_w(b)


@jax.jit
def _embed(idx, table):
    k = idx.shape[0] // (NW * GATHER)
    idx3 = idx.reshape(NW, k, GATHER)
    mesh = plsc.VectorSubcoreMesh(core_axis_name="c", subcore_axis_name="s")
    out = pl.kernel(
        _body,
        out_type=jax.ShapeDtypeStruct((idx.shape[0], DIM), jnp.float32),
        mesh=mesh,
        scratch_types=(
            [pltpu.VMEM((k, GATHER), jnp.int32),
             pltpu.VMEM((NBUF, GATHER, DIM), jnp.float32)]
            + [pltpu.SemaphoreType.DMA] * (2 * NBUF)
        ),
        compiler_params=pltpu.CompilerParams(use_tc_tiling_on_sc=False),
    )(idx3, table)
    return out


def kernel(input_ids, table):
    B, S = input_ids.shape
    flat = input_ids.reshape(B * S).astype(jnp.int32)
    out = _embed(flat, table)
    return out.reshape(B, S, DIM)
